# baseline (speedup 1.0000x reference)
import functools

import jax
import jax.numpy as jnp
from jax import lax
from jax.experimental import pallas as pl
from jax.experimental.pallas import tpu as pltpu
from jax.experimental.pallas import tpu_sc as plsc

_NC = 2
_NS = 16
_NW = _NC * _NS
_CH = 2000
_BN = 2000


def _tc_xw1(x0, w1):
    n, d = x0.shape
    f = w1.shape[1]
    bn = 2000

    def body(x_ref, w_ref, o_ref):
        o_ref[...] = jnp.dot(x_ref[...], w_ref[...],
                             preferred_element_type=jnp.float32)

    return pl.pallas_call(
        body,
        grid=(n // bn,),
        in_specs=[pl.BlockSpec((bn, d), lambda i: (i, 0)),
                  pl.BlockSpec((d, f), lambda i: (0, 0))],
        out_specs=pl.BlockSpec((bn, f), lambda i: (i, 0)),
        out_shape=jax.ShapeDtypeStruct((n, f), jnp.float32),
    )(x0, w1)


def _sc_gather_deg(xw1, idx2d, zeros_n, ones_ch):
    n, f = xw1.shape
    nrows, ch = idx2d.shape
    nch = nrows // _NW
    per_w = nch * ch
    t = nrows * ch
    mesh = plsc.VectorSubcoreMesh(core_axis_name="c", subcore_axis_name="s")

    @functools.partial(
        pl.kernel,
        out_type=(jax.ShapeDtypeStruct((t, f), jnp.float32),
                  jax.ShapeDtypeStruct((n // _BN, _NC, _BN), jnp.float32)),
        mesh=mesh,
        scratch_types=[
            pltpu.VMEM((nch, ch), jnp.int32),
            pltpu.VMEM((ch, f), jnp.float32),
            pltpu.VMEM((ch, f), jnp.float32),
            pltpu.VMEM((ch,), jnp.float32),
            pltpu.VMEM_SHARED((n,), jnp.float32),
            pltpu.SemaphoreType.DMA,
            pltpu.SemaphoreType.DMA,
            pltpu.SemaphoreType.DMA,
            pltpu.SemaphoreType.DMA,
        ],
        compiler_params=pltpu.CompilerParams(use_tc_tiling_on_sc=False),
    )
    def k(xw1_hbm, idx_hbm, zeros_hbm, ones_hbm, g_hbm, degp_hbm,
          idx_v, rows0, rows1, ones_v, deg_sh, g0, g1, w0, w1):
        c = lax.axis_index("c")
        s = lax.axis_index("s")
        wid = c * _NS + s

        @pl.when(s == 0)
        def _():
            pltpu.sync_copy(zeros_hbm, deg_sh)

        pltpu.sync_copy(ones_hbm, ones_v)
        pltpu.sync_copy(idx_hbm.at[pl.ds(wid * nch, nch)], idx_v)
        plsc.subcore_barrier()

        rows = [rows0, rows1]
        gsems = [g0, g1]
        wsems = [w0, w1]
        g_desc = [None] * nch
        w_desc = [None] * nch
        g_desc[0] = pltpu.async_copy(xw1_hbm.at[idx_v.at[0]], rows[0],
                                     gsems[0])
        for kk in range(nch):
            b = kk % 2
            g_desc[kk].wait()
            base = wid * per_w + kk * ch
            w_desc[kk] = pltpu.async_copy(rows[b],
                                          g_hbm.at[pl.ds(base, ch)],
                                          wsems[b])
            pltpu.sync_copy(ones_v, deg_sh.at[idx_v.at[kk]], add=True)
            if kk + 1 < nch:
                if kk >= 1:
                    w_desc[kk - 1].wait()
                g_desc[kk + 1] = pltpu.async_copy(
                    xw1_hbm.at[idx_v.at[kk + 1]], rows[1 - b],
                    gsems[1 - b])
        if nch >= 2:
            w_desc[nch - 2].wait()
        w_desc[nch - 1].wait()
        plsc.subcore_barrier()

        @pl.when(s == 0)
        def _():
            def out_body(j, carry):
                pltpu.sync_copy(deg_sh.at[pl.ds(j * _BN, _BN)],
                                degp_hbm.at[j, c])
                return carry
            lax.fori_loop(0, n // _BN, out_body, None)

    return k(xw1, idx2d, zeros_n, ones_ch)


def _sc_scatter(y, idx2d, zeros_nf):
    n, f = zeros_nf.shape
    nrows, ch = idx2d.shape
    nch = nrows // _NW
    per_w = nch * ch
    mesh = plsc.VectorSubcoreMesh(core_axis_name="c", subcore_axis_name="s")

    @functools.partial(
        pl.kernel,
        out_type=jax.ShapeDtypeStruct((_NC, n, f), jnp.float32),
        mesh=mesh,
        scratch_types=[
            pltpu.VMEM((nch, ch), jnp.int32),
            pltpu.VMEM((ch, f), jnp.float32),
            pltpu.VMEM((ch, f), jnp.float32),
            pltpu.VMEM_SHARED((n, f), jnp.float32),
            pltpu.SemaphoreType.DMA,
            pltpu.SemaphoreType.DMA,
        ],
        compiler_params=pltpu.CompilerParams(use_tc_tiling_on_sc=False),
    )
    def k(y_hbm, idx_hbm, zeros_hbm, aggp_hbm, idx_v, rows0, rows1,
          agg_sh, s0, s1):
        c = lax.axis_index("c")
        s = lax.axis_index("s")
        wid = c * _NS + s

        @pl.when(s == 0)
        def _():
            pltpu.sync_copy(zeros_hbm, agg_sh)

        pltpu.sync_copy(idx_hbm.at[pl.ds(wid * nch, nch)], idx_v)
        plsc.subcore_barrier()

        rows = [rows0, rows1]
        sems = [s0, s1]
        y_desc = [None] * nch
        ybase = s * per_w
        y_desc[0] = pltpu.async_copy(y_hbm.at[pl.ds(ybase, ch)], rows[0],
                                     sems[0])
        for kk in range(nch):
            b = kk % 2
            y_desc[kk].wait()
            if kk + 1 < nch:
                y_desc[kk + 1] = pltpu.async_copy(
                    y_hbm.at[pl.ds(ybase + (kk + 1) * ch, ch)],
                    rows[1 - b], sems[1 - b])
            pltpu.sync_copy(rows[b], agg_sh.at[idx_v.at[kk]], add=True)
        plsc.subcore_barrier()

        @pl.when(s == 0)
        def _():
            pltpu.sync_copy(agg_sh, aggp_hbm.at[c])

    return k(y, idx2d, zeros_nf)


def _tc_edge(g2, e0r, w0k, w2k):
    m8, c = e0r.shape
    bm = 4000

    def body(gs_ref, gd_ref, e0_ref, w0_ref, w2_ref, en_ref, y_ref):
        e = gs_ref[0] + gd_ref[0] + jnp.dot(
            e0_ref[...], w0_ref[...], preferred_element_type=jnp.float32)
        e = jnp.maximum(e, 0.0)
        en_ref[...] = e
        y_ref[...] = jnp.dot(e, w2_ref[...],
                             preferred_element_type=jnp.float32)

    return pl.pallas_call(
        body,
        grid=(m8 // bm,),
        in_specs=[
            pl.BlockSpec((1, bm, c), lambda i: (0, i, 0)),
            pl.BlockSpec((1, bm, c), lambda i: (1, i, 0)),
            pl.BlockSpec((bm, c), lambda i: (i, 0)),
            pl.BlockSpec((c, c), lambda i: (0, 0)),
            pl.BlockSpec((c, c), lambda i: (0, 0)),
        ],
        out_specs=[pl.BlockSpec((bm, c), lambda i: (i, 0)),
                   pl.BlockSpec((bm, c), lambda i: (i, 0))],
        out_shape=[jax.ShapeDtypeStruct((m8, c), jnp.float32),
                   jax.ShapeDtypeStruct((m8, c), jnp.float32)],
    )(g2, g2, e0r, w0k, w2k)


def _tc_node(aggp, degp, x0, w3, alpha):
    n, d = x0.shape
    f = aggp.shape[2]
    bn = _BN

    def body(aggp_ref, degp_ref, x0_ref, w3_ref, a_ref, xn_ref):
        agg = aggp_ref[0] + aggp_ref[1]
        degs = degp_ref[0]
        deg = degs[0:1, :] + degs[1:2, :]
        dinv = jnp.where(deg > 0, 1.0 / jnp.maximum(deg, 1.0), 0.0)
        ex = jnp.maximum(agg * jnp.transpose(dinv), 0.0)
        xn_ref[...] = jnp.maximum(
            a_ref[0, 0] * x0_ref[...] + jnp.dot(
                ex, w3_ref[...], preferred_element_type=jnp.float32),
            0.0)

    return pl.pallas_call(
        body,
        grid=(n // bn,),
        in_specs=[
            pl.BlockSpec((2, bn, f), lambda i: (0, i, 0)),
            pl.BlockSpec((1, 2, bn), lambda i: (i, 0, 0)),
            pl.BlockSpec((bn, d), lambda i: (i, 0)),
            pl.BlockSpec((f, d), lambda i: (0, 0)),
            pl.BlockSpec((1, 1), lambda i: (0, 0)),
        ],
        out_specs=pl.BlockSpec((bn, d), lambda i: (i, 0)),
        out_shape=jax.ShapeDtypeStruct((n, d), jnp.float32),
    )(aggp, degp, x0, w3, alpha)


def kernel(X0, E0, edge_index, W0, W1, W2, W3, alpha):
    n, d = X0.shape
    m, f = E0.shape
    idx2d = edge_index.astype(jnp.int32).reshape(2 * m // _CH, _CH)
    zeros_n = jnp.zeros((n,), jnp.float32)
    ones_ch = jnp.ones((_CH,), jnp.float32)
    zeros_nf = jnp.zeros((n, f), jnp.float32)

    r = 128 // f
    eye = jnp.eye(r, dtype=jnp.float32)
    w0k = jnp.kron(eye, W0)
    w2k = jnp.kron(eye, W2)

    xw1 = _tc_xw1(X0, W1)
    g, degp = _sc_gather_deg(xw1, idx2d, zeros_n, ones_ch)
    g2 = g.reshape(2, m // r, r * f)
    e_new_r, y_r = _tc_edge(g2, E0.reshape(m // r, r * f), w0k, w2k)
    aggp = _sc_scatter(y_r.reshape(m, f), idx2d, zeros_nf)
    x_new = _tc_node(aggp, degp, X0, W3, alpha)
    return (x_new, e_new_r.reshape(m, f))


# speedup vs baseline: 1.0105x; 1.0105x over previous
import functools

import jax
import jax.numpy as jnp
from jax import lax
from jax.experimental import pallas as pl
from jax.experimental.pallas import tpu as pltpu
from jax.experimental.pallas import tpu_sc as plsc

_NC = 2
_NS = 16
_NW = _NC * _NS
_CH = 2000
_BN = 2000


def _tc_xw1(x0, w1):
    n, d = x0.shape
    f = w1.shape[1]
    bn = 2000

    def body(x_ref, w_ref, o_ref):
        o_ref[...] = jnp.dot(x_ref[...], w_ref[...],
                             preferred_element_type=jnp.float32)

    return pl.pallas_call(
        body,
        grid=(n // bn,),
        in_specs=[pl.BlockSpec((bn, d), lambda i: (i, 0)),
                  pl.BlockSpec((d, f), lambda i: (0, 0))],
        out_specs=pl.BlockSpec((bn, f), lambda i: (i, 0)),
        out_shape=jax.ShapeDtypeStruct((n, f), jnp.float32),
    )(x0, w1)


def _sc_gather_deg(xw1, idx2d, zeros_n, ones_ch):
    n, f = xw1.shape
    nrows, ch = idx2d.shape
    nch = nrows // _NW
    per_w = nch * ch
    t = nrows * ch
    mesh = plsc.VectorSubcoreMesh(core_axis_name="c", subcore_axis_name="s")

    @functools.partial(
        pl.kernel,
        out_type=(jax.ShapeDtypeStruct((t, f), jnp.float32),
                  jax.ShapeDtypeStruct((n // _BN, _NC, _BN), jnp.float32)),
        mesh=mesh,
        scratch_types=[
            pltpu.VMEM((nch, ch), jnp.int32),
            pltpu.VMEM((ch, f), jnp.float32),
            pltpu.VMEM((ch, f), jnp.float32),
            pltpu.VMEM((ch,), jnp.float32),
            pltpu.VMEM_SHARED((n,), jnp.float32),
            pltpu.SemaphoreType.DMA,
            pltpu.SemaphoreType.DMA,
            pltpu.SemaphoreType.DMA,
            pltpu.SemaphoreType.DMA,
        ],
        compiler_params=pltpu.CompilerParams(use_tc_tiling_on_sc=False),
    )
    def k(xw1_hbm, idx_hbm, zeros_hbm, ones_hbm, g_hbm, degp_hbm,
          idx_v, rows0, rows1, ones_v, deg_sh, g0, g1, w0, w1):
        c = lax.axis_index("c")
        s = lax.axis_index("s")
        wid = c * _NS + s

        @pl.when(s == 0)
        def _():
            pltpu.sync_copy(zeros_hbm, deg_sh)

        pltpu.sync_copy(ones_hbm, ones_v)
        pltpu.sync_copy(idx_hbm.at[pl.ds(wid * nch, nch)], idx_v)
        plsc.subcore_barrier()

        rows = [rows0, rows1]
        gsems = [g0, g1]
        wsems = [w0, w1]
        g_desc = [None] * nch
        w_desc = [None] * nch
        g_desc[0] = pltpu.async_copy(xw1_hbm.at[idx_v.at[0]], rows[0],
                                     gsems[0])
        for kk in range(nch):
            b = kk % 2
            g_desc[kk].wait()
            base = wid * per_w + kk * ch
            w_desc[kk] = pltpu.async_copy(rows[b],
                                          g_hbm.at[pl.ds(base, ch)],
                                          wsems[b])
            pltpu.sync_copy(ones_v, deg_sh.at[idx_v.at[kk]], add=True)
            if kk + 1 < nch:
                if kk >= 1:
                    w_desc[kk - 1].wait()
                g_desc[kk + 1] = pltpu.async_copy(
                    xw1_hbm.at[idx_v.at[kk + 1]], rows[1 - b],
                    gsems[1 - b])
        if nch >= 2:
            w_desc[nch - 2].wait()
        w_desc[nch - 1].wait()
        plsc.subcore_barrier()

        @pl.when(s == 0)
        def _():
            def out_body(j, carry):
                pltpu.sync_copy(deg_sh.at[pl.ds(j * _BN, _BN)],
                                degp_hbm.at[j, c])
                return carry
            lax.fori_loop(0, n // _BN, out_body, None)

    return k(xw1, idx2d, zeros_n, ones_ch)


def _sc_scatter(y, idx2d, zeros_nf):
    n, f = zeros_nf.shape
    nrows, ch = idx2d.shape
    nch = nrows // _NW
    per_w = nch * ch
    mesh = plsc.VectorSubcoreMesh(core_axis_name="c", subcore_axis_name="s")

    @functools.partial(
        pl.kernel,
        out_type=jax.ShapeDtypeStruct((_NC, n, f), jnp.float32),
        mesh=mesh,
        scratch_types=[
            pltpu.VMEM((nch, ch), jnp.int32),
            pltpu.VMEM((ch, f), jnp.float32),
            pltpu.VMEM((ch, f), jnp.float32),
            pltpu.VMEM_SHARED((n, f), jnp.float32),
            pltpu.SemaphoreType.DMA,
            pltpu.SemaphoreType.DMA,
        ],
        compiler_params=pltpu.CompilerParams(use_tc_tiling_on_sc=False),
    )
    def k(y_hbm, idx_hbm, zeros_hbm, aggp_hbm, idx_v, rows0, rows1,
          agg_sh, s0, s1):
        c = lax.axis_index("c")
        s = lax.axis_index("s")
        wid = c * _NS + s

        @pl.when(s == 0)
        def _():
            pltpu.sync_copy(zeros_hbm, agg_sh)

        pltpu.sync_copy(idx_hbm.at[pl.ds(wid * nch, nch)], idx_v)
        plsc.subcore_barrier()

        rows = [rows0, rows1]
        sems = [s0, s1]
        y_desc = [None] * nch
        ybase = s * per_w
        y_desc[0] = pltpu.async_copy(y_hbm.at[pl.ds(ybase, ch)], rows[0],
                                     sems[0])
        for kk in range(nch):
            b = kk % 2
            y_desc[kk].wait()
            if kk + 1 < nch:
                y_desc[kk + 1] = pltpu.async_copy(
                    y_hbm.at[pl.ds(ybase + (kk + 1) * ch, ch)],
                    rows[1 - b], sems[1 - b])
            pltpu.sync_copy(rows[b], agg_sh.at[idx_v.at[kk]], add=True)
        plsc.subcore_barrier()

        @pl.when(s == 0)
        def _():
            pltpu.sync_copy(agg_sh, aggp_hbm.at[c])

    return k(y, idx2d, zeros_nf)


def _tc_edge(g2, e0r, w0k):
    m8, c = e0r.shape
    bm = 4000

    def body(gs_ref, gd_ref, e0_ref, w0_ref, en_ref):
        e = gs_ref[0] + gd_ref[0] + jnp.dot(
            e0_ref[...], w0_ref[...], preferred_element_type=jnp.float32)
        en_ref[...] = jnp.maximum(e, 0.0)

    return pl.pallas_call(
        body,
        grid=(m8 // bm,),
        in_specs=[
            pl.BlockSpec((1, bm, c), lambda i: (0, i, 0)),
            pl.BlockSpec((1, bm, c), lambda i: (1, i, 0)),
            pl.BlockSpec((bm, c), lambda i: (i, 0)),
            pl.BlockSpec((c, c), lambda i: (0, 0)),
        ],
        out_specs=pl.BlockSpec((bm, c), lambda i: (i, 0)),
        out_shape=jax.ShapeDtypeStruct((m8, c), jnp.float32),
    )(g2, g2, e0r, w0k)


def _tc_node(aggp, degp, x0, w2, w3, alpha):
    n, d = x0.shape
    f = aggp.shape[2]
    bn = _BN

    def body(aggp_ref, degp_ref, x0_ref, w2_ref, w3_ref, a_ref, xn_ref):
        agg = jnp.dot(aggp_ref[0] + aggp_ref[1], w2_ref[...],
                      preferred_element_type=jnp.float32)
        degs = degp_ref[0]
        deg = degs[0:1, :] + degs[1:2, :]
        dinv = jnp.where(deg > 0, 1.0 / jnp.maximum(deg, 1.0), 0.0)
        ex = jnp.maximum(agg * jnp.transpose(dinv), 0.0)
        xn_ref[...] = jnp.maximum(
            a_ref[0, 0] * x0_ref[...] + jnp.dot(
                ex, w3_ref[...], preferred_element_type=jnp.float32),
            0.0)

    return pl.pallas_call(
        body,
        grid=(n // bn,),
        in_specs=[
            pl.BlockSpec((2, bn, f), lambda i: (0, i, 0)),
            pl.BlockSpec((1, 2, bn), lambda i: (i, 0, 0)),
            pl.BlockSpec((bn, d), lambda i: (i, 0)),
            pl.BlockSpec((f, f), lambda i: (0, 0)),
            pl.BlockSpec((f, d), lambda i: (0, 0)),
            pl.BlockSpec((1, 1), lambda i: (0, 0)),
        ],
        out_specs=pl.BlockSpec((bn, d), lambda i: (i, 0)),
        out_shape=jax.ShapeDtypeStruct((n, d), jnp.float32),
    )(aggp, degp, x0, w2, w3, alpha)


def kernel(X0, E0, edge_index, W0, W1, W2, W3, alpha):
    n, d = X0.shape
    m, f = E0.shape
    idx2d = edge_index.astype(jnp.int32).reshape(2 * m // _CH, _CH)
    zeros_n = jnp.zeros((n,), jnp.float32)
    ones_ch = jnp.ones((_CH,), jnp.float32)
    zeros_nf = jnp.zeros((n, f), jnp.float32)

    r = 128 // f
    eye = jnp.eye(r, dtype=jnp.float32)
    w0k = jnp.kron(eye, W0)

    xw1 = _tc_xw1(X0, W1)
    g, degp = _sc_gather_deg(xw1, idx2d, zeros_n, ones_ch)
    g2 = g.reshape(2, m // r, r * f)
    e_new_r = _tc_edge(g2, E0.reshape(m // r, r * f), w0k)
    e_new = e_new_r.reshape(m, f)
    aggp = _sc_scatter(e_new, idx2d, zeros_nf)
    x_new = _tc_node(aggp, degp, X0, W2, W3, alpha)
    return (x_new, e_new)
